# baseline (speedup 1.0000x reference)
import functools

import jax
import jax.numpy as jnp
from jax import lax
from jax.experimental import pallas as pl
from jax.experimental.pallas import tpu as pltpu
from jax.experimental.pallas import tpu_sc as plsc

_B = 16384
_D = 32
_H = 64
_NC = 2
_NS = 16
_NW = _NC * _NS
_BPW = _B // _NW

_BLK = 2048


def _gather_pair(user_table, item_table, user_indices, item_indices):
    mesh = plsc.VectorSubcoreMesh(core_axis_name="c", subcore_axis_name="s")

    @functools.partial(
        pl.kernel,
        mesh=mesh,
        out_type=(
            jax.ShapeDtypeStruct((_B, _D), jnp.float32),
            jax.ShapeDtypeStruct((_B, _D), jnp.float32),
        ),
        scratch_types=[
            pltpu.VMEM((_BPW,), jnp.int32),
            pltpu.VMEM((_BPW,), jnp.int32),
            pltpu.VMEM((_BPW, _D), jnp.float32),
            pltpu.VMEM((_BPW, _D), jnp.float32),
            pltpu.SemaphoreType.DMA,
            pltpu.SemaphoreType.DMA,
        ],
        compiler_params=pltpu.CompilerParams(use_tc_tiling_on_sc=False),
    )
    def gather_kernel(ut_hbm, it_hbm, ui_hbm, ii_hbm, ou_hbm, oi_hbm,
                      uidx_v, iidx_v, urows_v, irows_v, usem, isem):
        wid = lax.axis_index("s") * _NC + lax.axis_index("c")
        base = wid * _BPW
        pltpu.sync_copy(ui_hbm.at[pl.ds(base, _BPW)], uidx_v)
        pltpu.sync_copy(ii_hbm.at[pl.ds(base, _BPW)], iidx_v)
        cu = pltpu.async_copy(ut_hbm.at[uidx_v], urows_v, usem)
        ci = pltpu.async_copy(it_hbm.at[iidx_v], irows_v, isem)
        cu.wait()
        ci.wait()
        pltpu.sync_copy(urows_v, ou_hbm.at[pl.ds(base, _BPW)])
        pltpu.sync_copy(irows_v, oi_hbm.at[pl.ds(base, _BPW)])

    return gather_kernel(user_table, item_table, user_indices, item_indices)


def _mlp_body(ue_ref, ie_ref, w1_ref, b1_ref, w2_ref, b2_ref, gb_ref, o_ref):
    h = jnp.dot(ue_ref[...], w1_ref[:_D, :], preferred_element_type=jnp.float32)
    h = h + jnp.dot(ie_ref[...], w1_ref[_D:, :], preferred_element_type=jnp.float32)
    h = jnp.maximum(h + b1_ref[...], 0.0)
    y = lax.dot_general(w2_ref[...], h, (((1,), (1,)), ((), ())),
                        preferred_element_type=jnp.float32)
    o_ref[...] = y + b2_ref[...] + gb_ref[...]


def _mlp(ue, ie, W1, b1, W2, b2, gb):
    out = pl.pallas_call(
        _mlp_body,
        grid=(_B // _BLK,),
        in_specs=[
            pl.BlockSpec((_BLK, _D), lambda i: (i, 0)),
            pl.BlockSpec((_BLK, _D), lambda i: (i, 0)),
            pl.BlockSpec((2 * _D, _H), lambda i: (0, 0)),
            pl.BlockSpec((1, _H), lambda i: (0, 0)),
            pl.BlockSpec((1, _H), lambda i: (0, 0)),
            pl.BlockSpec((1, 1), lambda i: (0, 0)),
            pl.BlockSpec((1, 1), lambda i: (0, 0)),
        ],
        out_specs=pl.BlockSpec((1, _BLK), lambda i: (0, i)),
        out_shape=jax.ShapeDtypeStruct((1, _B), jnp.float32),
    )(ue, ie, W1, b1.reshape(1, _H), W2.reshape(1, _H), b2.reshape(1, 1),
      gb.reshape(1, 1))
    return out.reshape(_B)


def kernel(user_table, item_table, W1, b1, W2, b2, global_bias,
           user_indices, item_indices):
    ue, ie = _gather_pair(user_table, item_table, user_indices, item_indices)
    return _mlp(ue, ie, W1, b1, W2, b2, global_bias)


# speedup vs baseline: 1.3959x; 1.3959x over previous
import functools

import jax
import jax.numpy as jnp
from jax import lax
from jax.experimental import pallas as pl
from jax.experimental.pallas import tpu as pltpu
from jax.experimental.pallas import tpu_sc as plsc

_B = 16384
_D = 32
_H = 64
_N = 1000000
_NC = 2
_NS = 16
_NW = _NC * _NS
_BPW = _B // _NW
_CH = 128

_QS = 1 << 18
_RB = 1024
_NBLK = _QS // _RB
_IN_BLKS = (_N + _RB - 1) // _RB
_BLK = 2048


def _repack_body(x0_ref, x1_ref, x2_ref, x3_ref, o_ref):
    parts = []
    for ref in (x0_ref, x1_ref, x2_ref, x3_ref):
        parts.append(ref[...].T)
    o_ref[...] = jnp.concatenate(parts, axis=1)


def _make_in_spec(q):
    off = q * _NBLK

    def index_map(i):
        return (0, jnp.minimum(i + off, _IN_BLKS - 1))

    return pl.BlockSpec((_D, _RB), index_map)


def _repack(table_t):
    return pl.pallas_call(
        _repack_body,
        grid=(_NBLK,),
        in_specs=[_make_in_spec(q) for q in range(4)],
        out_specs=pl.BlockSpec((_RB, 4 * _D), lambda i: (i, 0)),
        out_shape=jax.ShapeDtypeStruct((_QS, 4 * _D), jnp.float32),
        compiler_params=pltpu.CompilerParams(
            dimension_semantics=("parallel",)),
    )(table_t, table_t, table_t, table_t)


def _gather_sc(z, idxm):
    mesh = plsc.VectorSubcoreMesh(core_axis_name="c", subcore_axis_name="s")

    @functools.partial(
        pl.kernel,
        mesh=mesh,
        out_type=jax.ShapeDtypeStruct((_B, 4 * _D), jnp.float32),
        scratch_types=[
            pltpu.VMEM((_BPW,), jnp.int32),
            pltpu.VMEM((_CH, 4 * _D), jnp.float32),
            pltpu.VMEM((_CH, 4 * _D), jnp.float32),
            pltpu.SemaphoreType.DMA,
            pltpu.SemaphoreType.DMA,
        ],
    )
    def gather_kernel(z_hbm, i_hbm, o_hbm, idx_v, bufa, bufb, sema, semb):
        wid = lax.axis_index("s") * _NC + lax.axis_index("c")
        base = wid * _BPW
        pltpu.sync_copy(i_hbm.at[pl.ds(base, _BPW)], idx_v)
        c0 = pltpu.async_copy(z_hbm.at[idx_v.at[pl.ds(0 * _CH, _CH)]], bufa, sema)
        c1 = pltpu.async_copy(z_hbm.at[idx_v.at[pl.ds(1 * _CH, _CH)]], bufb, semb)
        c0.wait()
        pltpu.sync_copy(bufa, o_hbm.at[pl.ds(base + 0 * _CH, _CH)])
        c2 = pltpu.async_copy(z_hbm.at[idx_v.at[pl.ds(2 * _CH, _CH)]], bufa, sema)
        c1.wait()
        pltpu.sync_copy(bufb, o_hbm.at[pl.ds(base + 1 * _CH, _CH)])
        c3 = pltpu.async_copy(z_hbm.at[idx_v.at[pl.ds(3 * _CH, _CH)]], bufb, semb)
        c2.wait()
        pltpu.sync_copy(bufa, o_hbm.at[pl.ds(base + 2 * _CH, _CH)])
        c3.wait()
        pltpu.sync_copy(bufb, o_hbm.at[pl.ds(base + 3 * _CH, _CH)])

    return gather_kernel(z, idxm)


def _mlp_body(ue4_ref, ie4_ref, ui_ref, ii_ref, w1u_ref, w1i_ref, b1_ref,
              w2t_ref, b2_ref, gb_ref, o_ref):
    lane_grp = jax.lax.broadcasted_iota(jnp.int32, (_BLK, 4 * _D), 1) // _D
    gu = ui_ref[...] >> 18
    gi = ii_ref[...] >> 18
    zero = jnp.zeros((), jnp.float32)
    xu = jnp.where(lane_grp == gu, ue4_ref[...], zero)
    xi = jnp.where(lane_grp == gi, ie4_ref[...], zero)
    h = jnp.dot(xu, w1u_ref[...], preferred_element_type=jnp.float32)
    h = h + jnp.dot(xi, w1i_ref[...], preferred_element_type=jnp.float32)
    h = jnp.maximum(h + b1_ref[...], 0.0)
    y = lax.dot_general(w2t_ref[...], h, (((1,), (1,)), ((), ())),
                        preferred_element_type=jnp.float32)
    o_ref[...] = y + b2_ref[...] + gb_ref[...]


def _mlp(ue4, ie4, ui2, ii2, W1s_u, W1s_i, b1, W2t, b2, gb):
    out = pl.pallas_call(
        _mlp_body,
        grid=(_B // _BLK,),
        in_specs=[
            pl.BlockSpec((_BLK, 4 * _D), lambda i: (i, 0)),
            pl.BlockSpec((_BLK, 4 * _D), lambda i: (i, 0)),
            pl.BlockSpec((_BLK, 1), lambda i: (i, 0)),
            pl.BlockSpec((_BLK, 1), lambda i: (i, 0)),
            pl.BlockSpec((4 * _D, _H), lambda i: (0, 0)),
            pl.BlockSpec((4 * _D, _H), lambda i: (0, 0)),
            pl.BlockSpec((1, _H), lambda i: (0, 0)),
            pl.BlockSpec((1, _H), lambda i: (0, 0)),
            pl.BlockSpec((1, 1), lambda i: (0, 0)),
            pl.BlockSpec((1, 1), lambda i: (0, 0)),
        ],
        out_specs=pl.BlockSpec((1, _BLK), lambda i: (0, i)),
        out_shape=jax.ShapeDtypeStruct((1, _B), jnp.float32),
        compiler_params=pltpu.CompilerParams(
            dimension_semantics=("parallel",)),
    )(ue4, ie4, ui2, ii2, W1s_u, W1s_i, b1.reshape(1, _H), W2t,
      b2.reshape(1, 1), gb.reshape(1, 1))
    return out.reshape(_B)


def kernel(user_table, item_table, W1, b1, W2, b2, global_bias,
           user_indices, item_indices):
    zu = _repack(user_table.T)
    zi = _repack(item_table.T)
    ue4 = _gather_sc(zu, user_indices & (_QS - 1))
    ie4 = _gather_sc(zi, item_indices & (_QS - 1))
    W1s_u = jnp.concatenate([W1[:_D]] * 4, axis=0)
    W1s_i = jnp.concatenate([W1[_D:]] * 4, axis=0)
    return _mlp(ue4, ie4, user_indices.reshape(_B, 1),
                item_indices.reshape(_B, 1), W1s_u, W1s_i,
                b1, W2.reshape(1, _H), b2, global_bias)


# speedup vs baseline: 1.9684x; 1.4101x over previous
import functools

import jax
import jax.numpy as jnp
from jax import lax
from jax.experimental import pallas as pl
from jax.experimental.pallas import tpu as pltpu
from jax.experimental.pallas import tpu_sc as plsc

_B = 16384
_D = 32
_H = 64
_N = 1000000
_NC = 2
_NS = 16
_NW = _NC * _NS
_BPW = _B // _NW
_CH = 128

_QS = 1 << 18
_RB = 1024
_NBLK = _QS // _RB
_IN_BLKS = (_N + _RB - 1) // _RB
_BLK = 2048


def _repack_body(x0_ref, x1_ref, x2_ref, x3_ref, o_ref):
    x = jnp.concatenate(
        [x0_ref[...], x1_ref[...], x2_ref[...], x3_ref[...]], axis=0)
    xb = x.astype(jnp.bfloat16)
    row = jax.lax.broadcasted_iota(jnp.int32, (4 * _D, 4 * _D), 0)
    col = jax.lax.broadcasted_iota(jnp.int32, (4 * _D, 4 * _D), 1)
    ident = (row == col).astype(jnp.bfloat16)
    o_ref[...] = lax.dot_general(xb, ident, (((0,), (0,)), ((), ())),
                                 preferred_element_type=jnp.float32)


def _make_in_spec(q):
    off = q * _NBLK

    def index_map(i):
        return (0, jnp.minimum(i + off, _IN_BLKS - 1))

    return pl.BlockSpec((_D, _RB), index_map)


def _repack(table_t):
    return pl.pallas_call(
        _repack_body,
        grid=(_NBLK,),
        in_specs=[_make_in_spec(q) for q in range(4)],
        out_specs=pl.BlockSpec((_RB, 4 * _D), lambda i: (i, 0)),
        out_shape=jax.ShapeDtypeStruct((_QS, 4 * _D), jnp.float32),
        compiler_params=pltpu.CompilerParams(
            dimension_semantics=("parallel",)),
    )(table_t, table_t, table_t, table_t)


def _gather_sc(z, idxm):
    mesh = plsc.VectorSubcoreMesh(core_axis_name="c", subcore_axis_name="s")

    @functools.partial(
        pl.kernel,
        mesh=mesh,
        out_type=jax.ShapeDtypeStruct((_B, 4 * _D), jnp.float32),
        scratch_types=[
            pltpu.VMEM((_BPW,), jnp.int32),
            pltpu.VMEM((_CH, 4 * _D), jnp.float32),
            pltpu.VMEM((_CH, 4 * _D), jnp.float32),
            pltpu.SemaphoreType.DMA,
            pltpu.SemaphoreType.DMA,
        ],
    )
    def gather_kernel(z_hbm, i_hbm, o_hbm, idx_v, bufa, bufb, sema, semb):
        wid = lax.axis_index("s") * _NC + lax.axis_index("c")
        base = wid * _BPW
        pltpu.sync_copy(i_hbm.at[pl.ds(base, _BPW)], idx_v)
        c0 = pltpu.async_copy(z_hbm.at[idx_v.at[pl.ds(0 * _CH, _CH)]], bufa, sema)
        c1 = pltpu.async_copy(z_hbm.at[idx_v.at[pl.ds(1 * _CH, _CH)]], bufb, semb)
        c0.wait()
        pltpu.sync_copy(bufa, o_hbm.at[pl.ds(base + 0 * _CH, _CH)])
        c2 = pltpu.async_copy(z_hbm.at[idx_v.at[pl.ds(2 * _CH, _CH)]], bufa, sema)
        c1.wait()
        pltpu.sync_copy(bufb, o_hbm.at[pl.ds(base + 1 * _CH, _CH)])
        c3 = pltpu.async_copy(z_hbm.at[idx_v.at[pl.ds(3 * _CH, _CH)]], bufb, semb)
        c2.wait()
        pltpu.sync_copy(bufa, o_hbm.at[pl.ds(base + 2 * _CH, _CH)])
        c3.wait()
        pltpu.sync_copy(bufb, o_hbm.at[pl.ds(base + 3 * _CH, _CH)])

    return gather_kernel(z, idxm)


def _mlp_body(ue4_ref, ie4_ref, ui_ref, ii_ref, w1u_ref, w1i_ref, b1_ref,
              w2t_ref, b2_ref, gb_ref, o_ref):
    lane_grp = jax.lax.broadcasted_iota(jnp.int32, (_BLK, 4 * _D), 1) // _D
    gu = ui_ref[...] >> 18
    gi = ii_ref[...] >> 18
    zero = jnp.zeros((), jnp.float32)
    xu = jnp.where(lane_grp == gu, ue4_ref[...], zero)
    xi = jnp.where(lane_grp == gi, ie4_ref[...], zero)
    h = jnp.dot(xu, w1u_ref[...], preferred_element_type=jnp.float32)
    h = h + jnp.dot(xi, w1i_ref[...], preferred_element_type=jnp.float32)
    h = jnp.maximum(h + b1_ref[...], 0.0)
    y = lax.dot_general(w2t_ref[...], h, (((1,), (1,)), ((), ())),
                        preferred_element_type=jnp.float32)
    o_ref[...] = y + b2_ref[...] + gb_ref[...]


def _mlp(ue4, ie4, ui2, ii2, W1s_u, W1s_i, b1, W2t, b2, gb):
    out = pl.pallas_call(
        _mlp_body,
        grid=(_B // _BLK,),
        in_specs=[
            pl.BlockSpec((_BLK, 4 * _D), lambda i: (i, 0)),
            pl.BlockSpec((_BLK, 4 * _D), lambda i: (i, 0)),
            pl.BlockSpec((_BLK, 1), lambda i: (i, 0)),
            pl.BlockSpec((_BLK, 1), lambda i: (i, 0)),
            pl.BlockSpec((4 * _D, _H), lambda i: (0, 0)),
            pl.BlockSpec((4 * _D, _H), lambda i: (0, 0)),
            pl.BlockSpec((1, _H), lambda i: (0, 0)),
            pl.BlockSpec((1, _H), lambda i: (0, 0)),
            pl.BlockSpec((1, 1), lambda i: (0, 0)),
            pl.BlockSpec((1, 1), lambda i: (0, 0)),
        ],
        out_specs=pl.BlockSpec((1, _BLK), lambda i: (0, i)),
        out_shape=jax.ShapeDtypeStruct((1, _B), jnp.float32),
        compiler_params=pltpu.CompilerParams(
            dimension_semantics=("parallel",)),
    )(ue4, ie4, ui2, ii2, W1s_u, W1s_i, b1.reshape(1, _H), W2t,
      b2.reshape(1, 1), gb.reshape(1, 1))
    return out.reshape(_B)


def kernel(user_table, item_table, W1, b1, W2, b2, global_bias,
           user_indices, item_indices):
    zu = _repack(user_table.T)
    zi = _repack(item_table.T)
    ue4 = _gather_sc(zu, user_indices & (_QS - 1))
    ie4 = _gather_sc(zi, item_indices & (_QS - 1))
    W1s_u = jnp.concatenate([W1[:_D]] * 4, axis=0)
    W1s_i = jnp.concatenate([W1[_D:]] * 4, axis=0)
    return _mlp(ue4, ie4, user_indices.reshape(_B, 1),
                item_indices.reshape(_B, 1), W1s_u, W1s_i,
                b1, W2.reshape(1, _H), b2, global_bias)


# speedup vs baseline: 3.6810x; 1.8701x over previous
import functools

import jax
import jax.numpy as jnp
from jax import lax
from jax.experimental import pallas as pl
from jax.experimental.pallas import tpu as pltpu
from jax.experimental.pallas import tpu_sc as plsc

_B = 16384
_D = 32
_H = 64
_N = 1000000
_NC = 2
_NS = 16
_NW = _NC * _NS
_BPW = _B // _NW
_CH = 128

_QS = 1 << 18
_RB = 4096
_NBLK = _QS // _RB
_IN_BLKS = (_N + _RB - 1) // _RB
_BLK = 2048


def _repack_body(x0_ref, x1_ref, x2_ref, x3_ref, o_ref):
    x = jnp.concatenate(
        [x0_ref[...], x1_ref[...], x2_ref[...], x3_ref[...]], axis=0)
    xb = x.astype(jnp.bfloat16)
    row = jax.lax.broadcasted_iota(jnp.int32, (4 * _D, 4 * _D), 0)
    col = jax.lax.broadcasted_iota(jnp.int32, (4 * _D, 4 * _D), 1)
    ident = (row == col).astype(jnp.bfloat16)
    o_ref[...] = lax.dot_general(xb, ident, (((0,), (0,)), ((), ())),
                                 preferred_element_type=jnp.float32)


def _make_in_spec(q):
    off = q * _NBLK

    def index_map(i):
        return (0, jnp.minimum(i + off, _IN_BLKS - 1))

    return pl.BlockSpec((_D, _RB), index_map)


def _repack(table_t):
    return pl.pallas_call(
        _repack_body,
        grid=(_NBLK,),
        in_specs=[_make_in_spec(q) for q in range(4)],
        out_specs=pl.BlockSpec((_RB, 4 * _D), lambda i: (i, 0)),
        out_shape=jax.ShapeDtypeStruct((_QS, 4 * _D), jnp.float32),
        compiler_params=pltpu.CompilerParams(
            dimension_semantics=("parallel",)),
    )(table_t, table_t, table_t, table_t)


def _gather_sc(z, idxm):
    mesh = plsc.VectorSubcoreMesh(core_axis_name="c", subcore_axis_name="s")

    @functools.partial(
        pl.kernel,
        mesh=mesh,
        out_type=jax.ShapeDtypeStruct((_B, 4 * _D), jnp.float32),
        scratch_types=[
            pltpu.VMEM((_BPW,), jnp.int32),
            pltpu.VMEM((_CH, 4 * _D), jnp.float32),
            pltpu.VMEM((_CH, 4 * _D), jnp.float32),
            pltpu.SemaphoreType.DMA,
            pltpu.SemaphoreType.DMA,
        ],
    )
    def gather_kernel(z_hbm, i_hbm, o_hbm, idx_v, bufa, bufb, sema, semb):
        wid = lax.axis_index("s") * _NC + lax.axis_index("c")
        base = wid * _BPW
        pltpu.sync_copy(i_hbm.at[pl.ds(base, _BPW)], idx_v)
        c0 = pltpu.async_copy(z_hbm.at[idx_v.at[pl.ds(0 * _CH, _CH)]], bufa, sema)
        c1 = pltpu.async_copy(z_hbm.at[idx_v.at[pl.ds(1 * _CH, _CH)]], bufb, semb)
        c0.wait()
        pltpu.sync_copy(bufa, o_hbm.at[pl.ds(base + 0 * _CH, _CH)])
        c2 = pltpu.async_copy(z_hbm.at[idx_v.at[pl.ds(2 * _CH, _CH)]], bufa, sema)
        c1.wait()
        pltpu.sync_copy(bufb, o_hbm.at[pl.ds(base + 1 * _CH, _CH)])
        c3 = pltpu.async_copy(z_hbm.at[idx_v.at[pl.ds(3 * _CH, _CH)]], bufb, semb)
        c2.wait()
        pltpu.sync_copy(bufa, o_hbm.at[pl.ds(base + 2 * _CH, _CH)])
        c3.wait()
        pltpu.sync_copy(bufb, o_hbm.at[pl.ds(base + 3 * _CH, _CH)])

    return gather_kernel(z, idxm)


def _mlp_body(ue4_ref, ie4_ref, ui_ref, ii_ref, w1u_ref, w1i_ref, b1_ref,
              w2t_ref, b2_ref, gb_ref, o_ref):
    lane_grp = jax.lax.broadcasted_iota(jnp.int32, (_BLK, 4 * _D), 1) // _D
    gu = ui_ref[...] >> 18
    gi = ii_ref[...] >> 18
    zero = jnp.zeros((), jnp.float32)
    xu = jnp.where(lane_grp == gu, ue4_ref[...], zero)
    xi = jnp.where(lane_grp == gi, ie4_ref[...], zero)
    h = jnp.dot(xu, w1u_ref[...], preferred_element_type=jnp.float32)
    h = h + jnp.dot(xi, w1i_ref[...], preferred_element_type=jnp.float32)
    h = jnp.maximum(h + b1_ref[...], 0.0)
    y = lax.dot_general(w2t_ref[...], h, (((1,), (1,)), ((), ())),
                        preferred_element_type=jnp.float32)
    o_ref[...] = y + b2_ref[...] + gb_ref[...]


def _mlp(ue4, ie4, ui2, ii2, W1s_u, W1s_i, b1, W2t, b2, gb):
    out = pl.pallas_call(
        _mlp_body,
        grid=(_B // _BLK,),
        in_specs=[
            pl.BlockSpec((_BLK, 4 * _D), lambda i: (i, 0)),
            pl.BlockSpec((_BLK, 4 * _D), lambda i: (i, 0)),
            pl.BlockSpec((_BLK, 1), lambda i: (i, 0)),
            pl.BlockSpec((_BLK, 1), lambda i: (i, 0)),
            pl.BlockSpec((4 * _D, _H), lambda i: (0, 0)),
            pl.BlockSpec((4 * _D, _H), lambda i: (0, 0)),
            pl.BlockSpec((1, _H), lambda i: (0, 0)),
            pl.BlockSpec((1, _H), lambda i: (0, 0)),
            pl.BlockSpec((1, 1), lambda i: (0, 0)),
            pl.BlockSpec((1, 1), lambda i: (0, 0)),
        ],
        out_specs=pl.BlockSpec((1, _BLK), lambda i: (0, i)),
        out_shape=jax.ShapeDtypeStruct((1, _B), jnp.float32),
        compiler_params=pltpu.CompilerParams(
            dimension_semantics=("parallel",)),
    )(ue4, ie4, ui2, ii2, W1s_u, W1s_i, b1.reshape(1, _H), W2t,
      b2.reshape(1, 1), gb.reshape(1, 1))
    return out.reshape(_B)


def kernel(user_table, item_table, W1, b1, W2, b2, global_bias,
           user_indices, item_indices):
    zu = _repack(user_table.T)
    zi = _repack(item_table.T)
    ue4 = _gather_sc(zu, user_indices & (_QS - 1))
    ie4 = _gather_sc(zi, item_indices & (_QS - 1))
    W1s_u = jnp.concatenate([W1[:_D]] * 4, axis=0)
    W1s_i = jnp.concatenate([W1[_D:]] * 4, axis=0)
    return _mlp(ue4, ie4, user_indices.reshape(_B, 1),
                item_indices.reshape(_B, 1), W1s_u, W1s_i,
                b1, W2.reshape(1, _H), b2, global_bias)


# speedup vs baseline: 4.0347x; 1.0961x over previous
import functools

import jax
import jax.numpy as jnp
from jax import lax
from jax.experimental import pallas as pl
from jax.experimental.pallas import tpu as pltpu
from jax.experimental.pallas import tpu_sc as plsc

_B = 16384
_D = 32
_H = 64
_N = 1000000
_NC = 2
_NS = 16
_NW = _NC * _NS
_BPW = _B // _NW
_CH = 128

_QS = 1 << 18
_RB = 4096
_NBLK = _QS // _RB
_IN_BLKS = (_N + _RB - 1) // _RB
_BLK = 2048


def _repack_body(x0_ref, x1_ref, x2_ref, x3_ref, o_ref):
    step = pl.program_id(0)
    col = (3 * _QS + step * _RB
           + jax.lax.broadcasted_iota(jnp.int32, (_D, _RB), 1))
    x3 = jnp.where(col < _N, x3_ref[...], jnp.zeros((), jnp.float32))
    x = jnp.concatenate(
        [x0_ref[...], x1_ref[...], x2_ref[...], x3], axis=0)
    xb = x.astype(jnp.bfloat16)
    row = jax.lax.broadcasted_iota(jnp.int32, (4 * _D, 4 * _D), 0)
    col = jax.lax.broadcasted_iota(jnp.int32, (4 * _D, 4 * _D), 1)
    ident = (row == col).astype(jnp.bfloat16)
    y = lax.dot_general(xb, ident, (((0,), (0,)), ((), ())),
                        preferred_element_type=jnp.float32
                        ).astype(jnp.bfloat16)
    o_ref[...] = pltpu.bitcast(y, jnp.int32)


def _make_in_spec(q):
    off = q * _NBLK

    def index_map(i):
        return (0, jnp.minimum(i + off, _IN_BLKS - 1))

    return pl.BlockSpec((_D, _RB), index_map)


def _repack(table_t):
    return pl.pallas_call(
        _repack_body,
        grid=(_NBLK,),
        in_specs=[_make_in_spec(q) for q in range(4)],
        out_specs=pl.BlockSpec((_RB // 2, 4 * _D), lambda i: (i, 0)),
        out_shape=jax.ShapeDtypeStruct((_QS // 2, 4 * _D), jnp.int32),
        compiler_params=pltpu.CompilerParams(
            dimension_semantics=("parallel",)),
    )(table_t, table_t, table_t, table_t)


def _gather_sc(z, idxm):
    mesh = plsc.VectorSubcoreMesh(core_axis_name="c", subcore_axis_name="s")

    @functools.partial(
        pl.kernel,
        mesh=mesh,
        out_type=jax.ShapeDtypeStruct((_B, 4 * _D), jnp.int32),
        scratch_types=[
            pltpu.VMEM((_BPW,), jnp.int32),
            pltpu.VMEM((_CH, 4 * _D), jnp.int32),
            pltpu.VMEM((_CH, 4 * _D), jnp.int32),
            pltpu.SemaphoreType.DMA,
            pltpu.SemaphoreType.DMA,
        ],
    )
    def gather_kernel(z_hbm, i_hbm, o_hbm, idx_v, bufa, bufb, sema, semb):
        wid = lax.axis_index("s") * _NC + lax.axis_index("c")
        base = wid * _BPW
        pltpu.sync_copy(i_hbm.at[pl.ds(base, _BPW)], idx_v)
        c0 = pltpu.async_copy(z_hbm.at[idx_v.at[pl.ds(0 * _CH, _CH)]], bufa, sema)
        c1 = pltpu.async_copy(z_hbm.at[idx_v.at[pl.ds(1 * _CH, _CH)]], bufb, semb)
        c0.wait()
        pltpu.sync_copy(bufa, o_hbm.at[pl.ds(base + 0 * _CH, _CH)])
        c2 = pltpu.async_copy(z_hbm.at[idx_v.at[pl.ds(2 * _CH, _CH)]], bufa, sema)
        c1.wait()
        pltpu.sync_copy(bufb, o_hbm.at[pl.ds(base + 1 * _CH, _CH)])
        c3 = pltpu.async_copy(z_hbm.at[idx_v.at[pl.ds(3 * _CH, _CH)]], bufb, semb)
        c2.wait()
        pltpu.sync_copy(bufa, o_hbm.at[pl.ds(base + 2 * _CH, _CH)])
        c3.wait()
        pltpu.sync_copy(bufb, o_hbm.at[pl.ds(base + 3 * _CH, _CH)])

    return gather_kernel(z, idxm)


def _mlp_body(ue4_ref, ie4_ref, ui_ref, ii_ref, w1u_ref, w1i_ref, b1_ref,
              w2t_ref, b2_ref, gb_ref, o_ref):
    lane_grp = jax.lax.broadcasted_iota(jnp.int32, (_BLK, 4 * _D), 1) // _D
    zero = jnp.zeros((), jnp.float32)

    def unpack_select(w_ref, idx_ref):
        w = w_ref[...]
        g = idx_ref[...] >> 18
        p = idx_ref[...] & 1
        hi = lax.bitcast_convert_type(w & jnp.int32(-65536), jnp.float32)
        lo = lax.bitcast_convert_type(w << 16, jnp.float32)
        x = jnp.where(p == 0, lo, hi)
        return jnp.where(lane_grp == g, x, zero)

    xu = unpack_select(ue4_ref, ui_ref)
    xi = unpack_select(ie4_ref, ii_ref)
    h = jnp.dot(xu, w1u_ref[...], preferred_element_type=jnp.float32)
    h = h + jnp.dot(xi, w1i_ref[...], preferred_element_type=jnp.float32)
    h = jnp.maximum(h + b1_ref[...], 0.0)
    y = lax.dot_general(w2t_ref[...], h, (((1,), (1,)), ((), ())),
                        preferred_element_type=jnp.float32)
    o_ref[...] = y + b2_ref[...] + gb_ref[...]


def _mlp(ue4, ie4, ui2, ii2, W1s_u, W1s_i, b1, W2t, b2, gb):
    out = pl.pallas_call(
        _mlp_body,
        grid=(_B // _BLK,),
        in_specs=[
            pl.BlockSpec((_BLK, 4 * _D), lambda i: (i, 0)),
            pl.BlockSpec((_BLK, 4 * _D), lambda i: (i, 0)),
            pl.BlockSpec((_BLK, 1), lambda i: (i, 0)),
            pl.BlockSpec((_BLK, 1), lambda i: (i, 0)),
            pl.BlockSpec((4 * _D, _H), lambda i: (0, 0)),
            pl.BlockSpec((4 * _D, _H), lambda i: (0, 0)),
            pl.BlockSpec((1, _H), lambda i: (0, 0)),
            pl.BlockSpec((1, _H), lambda i: (0, 0)),
            pl.BlockSpec((1, 1), lambda i: (0, 0)),
            pl.BlockSpec((1, 1), lambda i: (0, 0)),
        ],
        out_specs=pl.BlockSpec((1, _BLK), lambda i: (0, i)),
        out_shape=jax.ShapeDtypeStruct((1, _B), jnp.float32),
        compiler_params=pltpu.CompilerParams(
            dimension_semantics=("parallel",)),
    )(ue4, ie4, ui2, ii2, W1s_u, W1s_i, b1.reshape(1, _H), W2t,
      b2.reshape(1, 1), gb.reshape(1, 1))
    return out.reshape(_B)


def kernel(user_table, item_table, W1, b1, W2, b2, global_bias,
           user_indices, item_indices):
    zu = _repack(user_table.T)
    zi = _repack(item_table.T)
    ue4 = _gather_sc(zu, (user_indices & (_QS - 1)) >> 1)
    ie4 = _gather_sc(zi, (item_indices & (_QS - 1)) >> 1)
    W1s_u = jnp.concatenate([W1[:_D]] * 4, axis=0)
    W1s_i = jnp.concatenate([W1[_D:]] * 4, axis=0)
    return _mlp(ue4, ie4, user_indices.reshape(_B, 1),
                item_indices.reshape(_B, 1), W1s_u, W1s_i,
                b1, W2.reshape(1, _H), b2, global_bias)


# speedup vs baseline: 4.9227x; 1.2201x over previous
import functools

import jax
import jax.numpy as jnp
from jax import lax
from jax.experimental import pallas as pl
from jax.experimental.pallas import tpu as pltpu
from jax.experimental.pallas import tpu_sc as plsc

_B = 16384
_D = 32
_H = 64
_N = 1000000
_NC = 2
_NS = 16
_NW = _NC * _NS
_BPW = _B // _NW
_CH = 128

_QS = 1 << 18
_RB = 8192
_NBLK = _QS // _RB
_IN_BLKS = (_N + _RB - 1) // _RB
_BLK = 4096


def _repack_body(x0_ref, x1_ref, x2_ref, x3_ref, o_ref):
    step = pl.program_id(0)
    col = (3 * _QS + step * _RB
           + jax.lax.broadcasted_iota(jnp.int32, (_D, _RB), 1))
    x3 = jnp.where(col < _N, x3_ref[...], jnp.zeros((), jnp.float32))
    x = jnp.concatenate(
        [x0_ref[...], x1_ref[...], x2_ref[...], x3], axis=0)
    xb = x.astype(jnp.bfloat16)
    row = jax.lax.broadcasted_iota(jnp.int32, (4 * _D, 4 * _D), 0)
    col = jax.lax.broadcasted_iota(jnp.int32, (4 * _D, 4 * _D), 1)
    ident = (row == col).astype(jnp.bfloat16)
    y = lax.dot_general(xb, ident, (((0,), (0,)), ((), ())),
                        preferred_element_type=jnp.float32
                        ).astype(jnp.bfloat16)
    o_ref[...] = pltpu.bitcast(y, jnp.int32)


def _make_in_spec(q):
    off = q * _NBLK

    def index_map(i):
        return (0, jnp.minimum(i + off, _IN_BLKS - 1))

    return pl.BlockSpec((_D, _RB), index_map)


def _repack(table_t):
    return pl.pallas_call(
        _repack_body,
        grid=(_NBLK,),
        in_specs=[_make_in_spec(q) for q in range(4)],
        out_specs=pl.BlockSpec((_RB // 2, 4 * _D), lambda i: (i, 0)),
        out_shape=jax.ShapeDtypeStruct((_QS // 2, 4 * _D), jnp.int32),
        compiler_params=pltpu.CompilerParams(
            dimension_semantics=("parallel",)),
    )(table_t, table_t, table_t, table_t)


def _gather_sc(z, idxm):
    mesh = plsc.VectorSubcoreMesh(core_axis_name="c", subcore_axis_name="s")

    @functools.partial(
        pl.kernel,
        mesh=mesh,
        out_type=jax.ShapeDtypeStruct((_B, 4 * _D), jnp.int32),
        scratch_types=[
            pltpu.VMEM((_BPW,), jnp.int32),
            pltpu.VMEM((_CH, 4 * _D), jnp.int32),
            pltpu.VMEM((_CH, 4 * _D), jnp.int32),
            pltpu.SemaphoreType.DMA,
            pltpu.SemaphoreType.DMA,
        ],
    )
    def gather_kernel(z_hbm, i_hbm, o_hbm, idx_v, bufa, bufb, sema, semb):
        wid = lax.axis_index("s") * _NC + lax.axis_index("c")
        base = wid * _BPW
        pltpu.sync_copy(i_hbm.at[pl.ds(base, _BPW)], idx_v)
        c0 = pltpu.async_copy(z_hbm.at[idx_v.at[pl.ds(0 * _CH, _CH)]], bufa, sema)
        c1 = pltpu.async_copy(z_hbm.at[idx_v.at[pl.ds(1 * _CH, _CH)]], bufb, semb)
        c0.wait()
        pltpu.sync_copy(bufa, o_hbm.at[pl.ds(base + 0 * _CH, _CH)])
        c2 = pltpu.async_copy(z_hbm.at[idx_v.at[pl.ds(2 * _CH, _CH)]], bufa, sema)
        c1.wait()
        pltpu.sync_copy(bufb, o_hbm.at[pl.ds(base + 1 * _CH, _CH)])
        c3 = pltpu.async_copy(z_hbm.at[idx_v.at[pl.ds(3 * _CH, _CH)]], bufb, semb)
        c2.wait()
        pltpu.sync_copy(bufa, o_hbm.at[pl.ds(base + 2 * _CH, _CH)])
        c3.wait()
        pltpu.sync_copy(bufb, o_hbm.at[pl.ds(base + 3 * _CH, _CH)])

    return gather_kernel(z, idxm)


def _mlp_body(ue4_ref, ie4_ref, ui_ref, ii_ref, w1u_ref, w1i_ref, b1_ref,
              w2t_ref, b2_ref, gb_ref, o_ref):
    lane_grp = jax.lax.broadcasted_iota(jnp.int32, (_BLK, 4 * _D), 1) // _D
    zero = jnp.zeros((), jnp.float32)

    def unpack_select(w_ref, idx_ref):
        w = w_ref[...]
        g = idx_ref[...] >> 18
        p = idx_ref[...] & 1
        hi = lax.bitcast_convert_type(w & jnp.int32(-65536), jnp.float32)
        lo = lax.bitcast_convert_type(w << 16, jnp.float32)
        x = jnp.where(p == 0, lo, hi)
        return jnp.where(lane_grp == g, x, zero)

    xu = unpack_select(ue4_ref, ui_ref)
    xi = unpack_select(ie4_ref, ii_ref)
    h = jnp.dot(xu, w1u_ref[...], preferred_element_type=jnp.float32)
    h = h + jnp.dot(xi, w1i_ref[...], preferred_element_type=jnp.float32)
    h = jnp.maximum(h + b1_ref[...], 0.0)
    y = lax.dot_general(w2t_ref[...], h, (((1,), (1,)), ((), ())),
                        preferred_element_type=jnp.float32)
    o_ref[...] = y + b2_ref[...] + gb_ref[...]


def _mlp(ue4, ie4, ui2, ii2, W1s_u, W1s_i, b1, W2t, b2, gb):
    out = pl.pallas_call(
        _mlp_body,
        grid=(_B // _BLK,),
        in_specs=[
            pl.BlockSpec((_BLK, 4 * _D), lambda i: (i, 0)),
            pl.BlockSpec((_BLK, 4 * _D), lambda i: (i, 0)),
            pl.BlockSpec((_BLK, 1), lambda i: (i, 0)),
            pl.BlockSpec((_BLK, 1), lambda i: (i, 0)),
            pl.BlockSpec((4 * _D, _H), lambda i: (0, 0)),
            pl.BlockSpec((4 * _D, _H), lambda i: (0, 0)),
            pl.BlockSpec((1, _H), lambda i: (0, 0)),
            pl.BlockSpec((1, _H), lambda i: (0, 0)),
            pl.BlockSpec((1, 1), lambda i: (0, 0)),
            pl.BlockSpec((1, 1), lambda i: (0, 0)),
        ],
        out_specs=pl.BlockSpec((1, _BLK), lambda i: (0, i)),
        out_shape=jax.ShapeDtypeStruct((1, _B), jnp.float32),
        compiler_params=pltpu.CompilerParams(
            dimension_semantics=("parallel",)),
    )(ue4, ie4, ui2, ii2, W1s_u, W1s_i, b1.reshape(1, _H), W2t,
      b2.reshape(1, 1), gb.reshape(1, 1))
    return out.reshape(_B)


def kernel(user_table, item_table, W1, b1, W2, b2, global_bias,
           user_indices, item_indices):
    zu = _repack(user_table.T)
    zi = _repack(item_table.T)
    ue4 = _gather_sc(zu, (user_indices & (_QS - 1)) >> 1)
    ie4 = _gather_sc(zi, (item_indices & (_QS - 1)) >> 1)
    W1s_u = jnp.concatenate([W1[:_D]] * 4, axis=0)
    W1s_i = jnp.concatenate([W1[_D:]] * 4, axis=0)
    return _mlp(ue4, ie4, user_indices.reshape(_B, 1),
                item_indices.reshape(_B, 1), W1s_u, W1s_i,
                b1, W2.reshape(1, _H), b2, global_bias)


# speedup vs baseline: 5.1553x; 1.0473x over previous
import functools

import jax
import jax.numpy as jnp
from jax import lax
from jax.experimental import pallas as pl
from jax.experimental.pallas import tpu as pltpu
from jax.experimental.pallas import tpu_sc as plsc

_B = 16384
_D = 32
_H = 64
_N = 1000000
_NC = 2
_NS = 16
_NW = _NC * _NS
_BPW = _B // _NW
_CH = 128

_QS = 1 << 18
_RB = 16384
_NBLK = _QS // _RB
_IN_BLKS = (_N + _RB - 1) // _RB
_BLK = 4096


def _repack_body(x0_ref, x1_ref, x2_ref, x3_ref, o_ref):
    step = pl.program_id(0)
    col = (3 * _QS + step * _RB
           + jax.lax.broadcasted_iota(jnp.int32, (_D, _RB), 1))
    x3 = jnp.where(col < _N, x3_ref[...], jnp.zeros((), jnp.float32))
    x = jnp.concatenate(
        [x0_ref[...], x1_ref[...], x2_ref[...], x3], axis=0)
    xb = x.astype(jnp.bfloat16)
    row = jax.lax.broadcasted_iota(jnp.int32, (4 * _D, 4 * _D), 0)
    col = jax.lax.broadcasted_iota(jnp.int32, (4 * _D, 4 * _D), 1)
    ident = (row == col).astype(jnp.bfloat16)
    y = lax.dot_general(xb, ident, (((0,), (0,)), ((), ())),
                        preferred_element_type=jnp.float32
                        ).astype(jnp.bfloat16)
    o_ref[...] = pltpu.bitcast(y, jnp.int32)


def _make_in_spec(q):
    off = q * _NBLK

    def index_map(i):
        return (0, jnp.minimum(i + off, _IN_BLKS - 1))

    return pl.BlockSpec((_D, _RB), index_map)


def _repack(table_t):
    return pl.pallas_call(
        _repack_body,
        grid=(_NBLK,),
        in_specs=[_make_in_spec(q) for q in range(4)],
        out_specs=pl.BlockSpec((_RB // 2, 4 * _D), lambda i: (i, 0)),
        out_shape=jax.ShapeDtypeStruct((_QS // 2, 4 * _D), jnp.int32),
        compiler_params=pltpu.CompilerParams(
            dimension_semantics=("parallel",)),
    )(table_t, table_t, table_t, table_t)


def _gather_sc(z, idxm):
    mesh = plsc.VectorSubcoreMesh(core_axis_name="c", subcore_axis_name="s")

    @functools.partial(
        pl.kernel,
        mesh=mesh,
        out_type=jax.ShapeDtypeStruct((_B, 4 * _D), jnp.int32),
        scratch_types=[
            pltpu.VMEM((_BPW,), jnp.int32),
            pltpu.VMEM((_CH, 4 * _D), jnp.int32),
            pltpu.VMEM((_CH, 4 * _D), jnp.int32),
            pltpu.SemaphoreType.DMA,
            pltpu.SemaphoreType.DMA,
        ],
    )
    def gather_kernel(z_hbm, i_hbm, o_hbm, idx_v, bufa, bufb, sema, semb):
        wid = lax.axis_index("s") * _NC + lax.axis_index("c")
        base = wid * _BPW
        pltpu.sync_copy(i_hbm.at[pl.ds(base, _BPW)], idx_v)
        c0 = pltpu.async_copy(z_hbm.at[idx_v.at[pl.ds(0 * _CH, _CH)]], bufa, sema)
        c1 = pltpu.async_copy(z_hbm.at[idx_v.at[pl.ds(1 * _CH, _CH)]], bufb, semb)
        c0.wait()
        pltpu.sync_copy(bufa, o_hbm.at[pl.ds(base + 0 * _CH, _CH)])
        c2 = pltpu.async_copy(z_hbm.at[idx_v.at[pl.ds(2 * _CH, _CH)]], bufa, sema)
        c1.wait()
        pltpu.sync_copy(bufb, o_hbm.at[pl.ds(base + 1 * _CH, _CH)])
        c3 = pltpu.async_copy(z_hbm.at[idx_v.at[pl.ds(3 * _CH, _CH)]], bufb, semb)
        c2.wait()
        pltpu.sync_copy(bufa, o_hbm.at[pl.ds(base + 2 * _CH, _CH)])
        c3.wait()
        pltpu.sync_copy(bufb, o_hbm.at[pl.ds(base + 3 * _CH, _CH)])

    return gather_kernel(z, idxm)


def _mlp_body(ue4_ref, ie4_ref, ui_ref, ii_ref, w1u_ref, w1i_ref, b1_ref,
              w2t_ref, b2_ref, gb_ref, o_ref):
    lane_grp = jax.lax.broadcasted_iota(jnp.int32, (_BLK, 4 * _D), 1) // _D
    zero = jnp.zeros((), jnp.float32)

    def unpack_select(w_ref, idx_ref):
        w = w_ref[...]
        g = idx_ref[...] >> 18
        p = idx_ref[...] & 1
        hi = lax.bitcast_convert_type(w & jnp.int32(-65536), jnp.float32)
        lo = lax.bitcast_convert_type(w << 16, jnp.float32)
        x = jnp.where(p == 0, lo, hi)
        return jnp.where(lane_grp == g, x, zero)

    xu = unpack_select(ue4_ref, ui_ref)
    xi = unpack_select(ie4_ref, ii_ref)
    h = jnp.dot(xu, w1u_ref[...], preferred_element_type=jnp.float32)
    h = h + jnp.dot(xi, w1i_ref[...], preferred_element_type=jnp.float32)
    h = jnp.maximum(h + b1_ref[...], 0.0)
    y = lax.dot_general(w2t_ref[...], h, (((1,), (1,)), ((), ())),
                        preferred_element_type=jnp.float32)
    o_ref[...] = y + b2_ref[...] + gb_ref[...]


def _mlp(ue4, ie4, ui2, ii2, W1s_u, W1s_i, b1, W2t, b2, gb):
    out = pl.pallas_call(
        _mlp_body,
        grid=(_B // _BLK,),
        in_specs=[
            pl.BlockSpec((_BLK, 4 * _D), lambda i: (i, 0)),
            pl.BlockSpec((_BLK, 4 * _D), lambda i: (i, 0)),
            pl.BlockSpec((_BLK, 1), lambda i: (i, 0)),
            pl.BlockSpec((_BLK, 1), lambda i: (i, 0)),
            pl.BlockSpec((4 * _D, _H), lambda i: (0, 0)),
            pl.BlockSpec((4 * _D, _H), lambda i: (0, 0)),
            pl.BlockSpec((1, _H), lambda i: (0, 0)),
            pl.BlockSpec((1, _H), lambda i: (0, 0)),
            pl.BlockSpec((1, 1), lambda i: (0, 0)),
            pl.BlockSpec((1, 1), lambda i: (0, 0)),
        ],
        out_specs=pl.BlockSpec((1, _BLK), lambda i: (0, i)),
        out_shape=jax.ShapeDtypeStruct((1, _B), jnp.float32),
        compiler_params=pltpu.CompilerParams(
            dimension_semantics=("parallel",)),
    )(ue4, ie4, ui2, ii2, W1s_u, W1s_i, b1.reshape(1, _H), W2t,
      b2.reshape(1, 1), gb.reshape(1, 1))
    return out.reshape(_B)


def kernel(user_table, item_table, W1, b1, W2, b2, global_bias,
           user_indices, item_indices):
    zu = _repack(user_table.T)
    zi = _repack(item_table.T)
    ue4 = _gather_sc(zu, (user_indices & (_QS - 1)) >> 1)
    ie4 = _gather_sc(zi, (item_indices & (_QS - 1)) >> 1)
    W1s_u = jnp.concatenate([W1[:_D]] * 4, axis=0)
    W1s_i = jnp.concatenate([W1[_D:]] * 4, axis=0)
    return _mlp(ue4, ie4, user_indices.reshape(_B, 1),
                item_indices.reshape(_B, 1), W1s_u, W1s_i,
                b1, W2.reshape(1, _H), b2, global_bias)


# speedup vs baseline: 5.1701x; 1.0029x over previous
import functools

import jax
import jax.numpy as jnp
from jax import lax
from jax.experimental import pallas as pl
from jax.experimental.pallas import tpu as pltpu
from jax.experimental.pallas import tpu_sc as plsc

_B = 16384
_D = 32
_H = 64
_N = 1000000
_NC = 2
_NS = 16
_NW = _NC * _NS
_BPW = _B // _NW
_CH = 128

_QS = 1 << 18
_RB = 16384
_NBLK = _QS // _RB
_IN_BLKS = (_N + _RB - 1) // _RB
_BLK = 4096


def _repack_body(x0_ref, x1_ref, x2_ref, x3_ref, o_ref):
    step = pl.program_id(0)
    col = (3 * _QS + step * _RB
           + jax.lax.broadcasted_iota(jnp.int32, (_D, _RB), 1))
    x3 = jnp.where(col < _N, x3_ref[...], jnp.zeros((), jnp.float32))
    x = jnp.concatenate(
        [x0_ref[...], x1_ref[...], x2_ref[...], x3], axis=0)
    xb = x.astype(jnp.bfloat16)
    row = jax.lax.broadcasted_iota(jnp.int32, (4 * _D, 4 * _D), 0)
    col = jax.lax.broadcasted_iota(jnp.int32, (4 * _D, 4 * _D), 1)
    ident = (row == col).astype(jnp.bfloat16)
    y = lax.dot_general(xb, ident, (((0,), (0,)), ((), ())),
                        preferred_element_type=jnp.float32
                        ).astype(jnp.bfloat16)
    o_ref[...] = pltpu.bitcast(y, jnp.int32)


def _make_in_spec(q):
    off = q * _NBLK

    def index_map(i):
        return (0, jnp.minimum(i + off, _IN_BLKS - 1))

    return pl.BlockSpec((_D, _RB), index_map)


def _repack(table_t):
    return pl.pallas_call(
        _repack_body,
        grid=(_NBLK,),
        in_specs=[_make_in_spec(q) for q in range(4)],
        out_specs=pl.BlockSpec((_RB // 2, 4 * _D), lambda i: (i, 0)),
        out_shape=jax.ShapeDtypeStruct((_QS // 2, 4 * _D), jnp.int32),
        compiler_params=pltpu.CompilerParams(
            dimension_semantics=("parallel",)),
    )(table_t, table_t, table_t, table_t)


def _gather_sc(z, idxm):
    mesh = plsc.VectorSubcoreMesh(core_axis_name="c", subcore_axis_name="s")

    @functools.partial(
        pl.kernel,
        mesh=mesh,
        out_type=jax.ShapeDtypeStruct((_B, 4 * _D), jnp.int32),
        scratch_types=[
            pltpu.VMEM((_BPW,), jnp.int32),
            pltpu.VMEM((_CH, 4 * _D), jnp.int32),
            pltpu.VMEM((_CH, 4 * _D), jnp.int32),
            pltpu.VMEM((_CH, 4 * _D), jnp.int32),
            pltpu.VMEM((_CH, 4 * _D), jnp.int32),
            pltpu.SemaphoreType.DMA,
            pltpu.SemaphoreType.DMA,
            pltpu.SemaphoreType.DMA,
            pltpu.SemaphoreType.DMA,
            pltpu.SemaphoreType.DMA,
        ],
    )
    def gather_kernel(z_hbm, i_hbm, o_hbm, idx_v, b0, b1, b2, b3,
                      s0, s1, s2, s3, so):
        wid = lax.axis_index("s") * _NC + lax.axis_index("c")
        base = wid * _BPW
        pltpu.sync_copy(i_hbm.at[pl.ds(base, _BPW)], idx_v)
        bufs = (b0, b1, b2, b3)
        sems = (s0, s1, s2, s3)
        cps = [pltpu.async_copy(z_hbm.at[idx_v.at[pl.ds(k * _CH, _CH)]],
                                bufs[k], sems[k]) for k in range(4)]
        outs = []
        for k in range(4):
            cps[k].wait()
            outs.append(pltpu.async_copy(
                bufs[k], o_hbm.at[pl.ds(base + k * _CH, _CH)], so))
        for k in range(4):
            outs[k].wait()

    return gather_kernel(z, idxm)


def _mlp_body(ue4_ref, ie4_ref, ui_ref, ii_ref, w1u_ref, w1i_ref, b1_ref,
              w2t_ref, b2_ref, gb_ref, o_ref):
    lane_grp = jax.lax.broadcasted_iota(jnp.int32, (_BLK, 4 * _D), 1) // _D
    zero = jnp.zeros((), jnp.float32)

    def unpack_select(w_ref, idx_ref):
        w = w_ref[...]
        g = idx_ref[...] >> 18
        p = idx_ref[...] & 1
        hi = lax.bitcast_convert_type(w & jnp.int32(-65536), jnp.float32)
        lo = lax.bitcast_convert_type(w << 16, jnp.float32)
        x = jnp.where(p == 0, lo, hi)
        return jnp.where(lane_grp == g, x, zero)

    xu = unpack_select(ue4_ref, ui_ref)
    xi = unpack_select(ie4_ref, ii_ref)
    h = jnp.dot(xu, w1u_ref[...], preferred_element_type=jnp.float32)
    h = h + jnp.dot(xi, w1i_ref[...], preferred_element_type=jnp.float32)
    h = jnp.maximum(h + b1_ref[...], 0.0)
    y = lax.dot_general(w2t_ref[...], h, (((1,), (1,)), ((), ())),
                        preferred_element_type=jnp.float32)
    o_ref[...] = y + b2_ref[...] + gb_ref[...]


def _mlp(ue4, ie4, ui2, ii2, W1s_u, W1s_i, b1, W2t, b2, gb):
    out = pl.pallas_call(
        _mlp_body,
        grid=(_B // _BLK,),
        in_specs=[
            pl.BlockSpec((_BLK, 4 * _D), lambda i: (i, 0)),
            pl.BlockSpec((_BLK, 4 * _D), lambda i: (i, 0)),
            pl.BlockSpec((_BLK, 1), lambda i: (i, 0)),
            pl.BlockSpec((_BLK, 1), lambda i: (i, 0)),
            pl.BlockSpec((4 * _D, _H), lambda i: (0, 0)),
            pl.BlockSpec((4 * _D, _H), lambda i: (0, 0)),
            pl.BlockSpec((1, _H), lambda i: (0, 0)),
            pl.BlockSpec((1, _H), lambda i: (0, 0)),
            pl.BlockSpec((1, 1), lambda i: (0, 0)),
            pl.BlockSpec((1, 1), lambda i: (0, 0)),
        ],
        out_specs=pl.BlockSpec((1, _BLK), lambda i: (0, i)),
        out_shape=jax.ShapeDtypeStruct((1, _B), jnp.float32),
        compiler_params=pltpu.CompilerParams(
            dimension_semantics=("parallel",)),
    )(ue4, ie4, ui2, ii2, W1s_u, W1s_i, b1.reshape(1, _H), W2t,
      b2.reshape(1, 1), gb.reshape(1, 1))
    return out.reshape(_B)


def kernel(user_table, item_table, W1, b1, W2, b2, global_bias,
           user_indices, item_indices):
    zu = _repack(user_table.T)
    zi = _repack(item_table.T)
    ue4 = _gather_sc(zu, (user_indices & (_QS - 1)) >> 1)
    ie4 = _gather_sc(zi, (item_indices & (_QS - 1)) >> 1)
    W1s_u = jnp.concatenate([W1[:_D]] * 4, axis=0)
    W1s_i = jnp.concatenate([W1[_D:]] * 4, axis=0)
    return _mlp(ue4, ie4, user_indices.reshape(_B, 1),
                item_indices.reshape(_B, 1), W1s_u, W1s_i,
                b1, W2.reshape(1, _H), b2, global_bias)


# speedup vs baseline: 5.1731x; 1.0006x over previous
import functools

import jax
import jax.numpy as jnp
from jax import lax
from jax.experimental import pallas as pl
from jax.experimental.pallas import tpu as pltpu
from jax.experimental.pallas import tpu_sc as plsc

_B = 16384
_D = 32
_H = 64
_N = 1000000
_NC = 2
_NS = 16
_NW = _NC * _NS
_BPW = _B // _NW
_CH = 128

_QS = 1 << 18
_RB = 16384
_NBLK = _QS // _RB
_IN_BLKS = (_N + _RB - 1) // _RB
_BLK = 4096


def _repack_body(x0_ref, x1_ref, x2_ref, x3_ref, o_ref):
    step = pl.program_id(0)
    col = (3 * _QS + step * _RB
           + jax.lax.broadcasted_iota(jnp.int32, (_D, _RB), 1))
    x3 = jnp.where(col < _N, x3_ref[...], jnp.zeros((), jnp.float32))
    x = jnp.concatenate(
        [x0_ref[...], x1_ref[...], x2_ref[...], x3], axis=0)
    xb = x.astype(jnp.bfloat16)
    row = jax.lax.broadcasted_iota(jnp.int32, (4 * _D, 4 * _D), 0)
    col = jax.lax.broadcasted_iota(jnp.int32, (4 * _D, 4 * _D), 1)
    ident = (row == col).astype(jnp.bfloat16)
    y = lax.dot_general(xb, ident, (((0,), (0,)), ((), ())),
                        preferred_element_type=jnp.float32
                        ).astype(jnp.bfloat16)
    o_ref[...] = pltpu.bitcast(y, jnp.int32)


def _make_in_spec(q):
    off = q * _NBLK

    def index_map(i):
        return (0, jnp.minimum(i + off, _IN_BLKS - 1))

    return pl.BlockSpec((_D, _RB), index_map)


def _repack(table_t):
    return pl.pallas_call(
        _repack_body,
        grid=(_NBLK,),
        in_specs=[_make_in_spec(q) for q in range(4)],
        out_specs=pl.BlockSpec((_RB // 2, 4 * _D), lambda i: (i, 0)),
        out_shape=jax.ShapeDtypeStruct((_QS // 2, 4 * _D), jnp.int32),
        compiler_params=pltpu.CompilerParams(
            dimension_semantics=("parallel",)),
    )(table_t, table_t, table_t, table_t)


def _gather_sc(z, idxm):
    mesh = plsc.VectorSubcoreMesh(core_axis_name="c", subcore_axis_name="s")

    @functools.partial(
        pl.kernel,
        mesh=mesh,
        out_type=jax.ShapeDtypeStruct((_B, 4 * _D), jnp.int32),
        scratch_types=[
            pltpu.VMEM((_BPW,), jnp.int32),
            pltpu.VMEM((_BPW, 4 * _D), jnp.int32),
            pltpu.SemaphoreType.DMA,
        ],
    )
    def gather_kernel(z_hbm, i_hbm, o_hbm, idx_v, rows_v, sem):
        wid = lax.axis_index("s") * _NC + lax.axis_index("c")
        base = wid * _BPW
        pltpu.sync_copy(i_hbm.at[pl.ds(base, _BPW)], idx_v)
        pltpu.async_copy(z_hbm.at[idx_v], rows_v, sem).wait()
        pltpu.sync_copy(rows_v, o_hbm.at[pl.ds(base, _BPW)])

    return gather_kernel(z, idxm)


def _mlp_body(ue4_ref, ie4_ref, ui_ref, ii_ref, w1u_ref, w1i_ref, b1_ref,
              w2t_ref, b2_ref, gb_ref, o_ref):
    lane_grp = jax.lax.broadcasted_iota(jnp.int32, (_BLK, 4 * _D), 1) // _D
    zero = jnp.zeros((), jnp.float32)

    def unpack_select(w_ref, idx_ref):
        w = w_ref[...]
        g = idx_ref[...] >> 18
        p = idx_ref[...] & 1
        hi = lax.bitcast_convert_type(w & jnp.int32(-65536), jnp.float32)
        lo = lax.bitcast_convert_type(w << 16, jnp.float32)
        x = jnp.where(p == 0, lo, hi)
        return jnp.where(lane_grp == g, x, zero)

    xu = unpack_select(ue4_ref, ui_ref)
    xi = unpack_select(ie4_ref, ii_ref)
    h = jnp.dot(xu, w1u_ref[...], preferred_element_type=jnp.float32)
    h = h + jnp.dot(xi, w1i_ref[...], preferred_element_type=jnp.float32)
    h = jnp.maximum(h + b1_ref[...], 0.0)
    y = lax.dot_general(w2t_ref[...], h, (((1,), (1,)), ((), ())),
                        preferred_element_type=jnp.float32)
    o_ref[...] = y + b2_ref[...] + gb_ref[...]


def _mlp(ue4, ie4, ui2, ii2, W1s_u, W1s_i, b1, W2t, b2, gb):
    out = pl.pallas_call(
        _mlp_body,
        grid=(_B // _BLK,),
        in_specs=[
            pl.BlockSpec((_BLK, 4 * _D), lambda i: (i, 0)),
            pl.BlockSpec((_BLK, 4 * _D), lambda i: (i, 0)),
            pl.BlockSpec((_BLK, 1), lambda i: (i, 0)),
            pl.BlockSpec((_BLK, 1), lambda i: (i, 0)),
            pl.BlockSpec((4 * _D, _H), lambda i: (0, 0)),
            pl.BlockSpec((4 * _D, _H), lambda i: (0, 0)),
            pl.BlockSpec((1, _H), lambda i: (0, 0)),
            pl.BlockSpec((1, _H), lambda i: (0, 0)),
            pl.BlockSpec((1, 1), lambda i: (0, 0)),
            pl.BlockSpec((1, 1), lambda i: (0, 0)),
        ],
        out_specs=pl.BlockSpec((1, _BLK), lambda i: (0, i)),
        out_shape=jax.ShapeDtypeStruct((1, _B), jnp.float32),
        compiler_params=pltpu.CompilerParams(
            dimension_semantics=("parallel",)),
    )(ue4, ie4, ui2, ii2, W1s_u, W1s_i, b1.reshape(1, _H), W2t,
      b2.reshape(1, 1), gb.reshape(1, 1))
    return out.reshape(_B)


def kernel(user_table, item_table, W1, b1, W2, b2, global_bias,
           user_indices, item_indices):
    zu = _repack(user_table.T)
    zi = _repack(item_table.T)
    ue4 = _gather_sc(zu, (user_indices & (_QS - 1)) >> 1)
    ie4 = _gather_sc(zi, (item_indices & (_QS - 1)) >> 1)
    W1s_u = jnp.concatenate([W1[:_D]] * 4, axis=0)
    W1s_i = jnp.concatenate([W1[_D:]] * 4, axis=0)
    return _mlp(ue4, ie4, user_indices.reshape(_B, 1),
                item_indices.reshape(_B, 1), W1s_u, W1s_i,
                b1, W2.reshape(1, _H), b2, global_bias)
